# baseline (speedup 1.0000x reference)
import functools

import jax
import jax.numpy as jnp
from jax import lax
from jax.experimental import pallas as pl
from jax.experimental.pallas import tpu as pltpu
from jax.experimental.pallas import tpu_sc as plsc

NW = 32
RSUB = 8
LANE_BLK = 256


def _weights_body(s_ref, w_ref, *, S):
    s = s_ref[...]
    c = jnp.zeros(s.shape, jnp.float32)
    for j in range(S):
        c = c + (s == s[j : j + 1, :]).astype(jnp.float32)
    w = 1.0 / c
    w_ref[: S, :] = w
    w_ref[S : S + 1, :] = 1.0 / jnp.sum(w, axis=0, keepdims=True)


def _sc_body(table_h, idx_h, self_h, w_h, out_h,
             idx_v, self_v, w_v,
             rows0, rows1, srows0, srows1, out0, out1,
             sem0, sem1, osem0, osem1,
             *, K, D, rows_c0, rows_c1):
    c = lax.axis_index("c")
    s = lax.axis_index("s")
    rows_mine = lax.select(c == 0, rows_c0, rows_c1)
    base = pl.multiple_of(c * (16 * rows_c0) + s * rows_mine, 2 * RSUB)
    rows_max = max(rows_c0, rows_c1)
    s_pad = -(-(K + 2) // 16) * 16
    pltpu.sync_copy(idx_h.at[pl.ds(base * K, rows_max * K)],
                    idx_v.at[pl.ds(0, rows_max * K)])
    pltpu.sync_copy(self_h.at[pl.ds(base, rows_max)],
                    self_v.at[pl.ds(0, rows_max)])
    pltpu.sync_copy(w_h.at[pl.ds(base * s_pad, rows_max * s_pad)],
                    w_v.at[pl.ds(0, rows_max * s_pad)])

    nvec = D // 16
    nstream = (RSUB * K) // 128
    nsteps = rows_mine // RSUB
    nouter = nsteps // 2

    def issue(t, rows_b, srows_b, sem_b):
        for g in range(nstream):
            pltpu.async_copy(
                table_h.at[idx_v.at[pl.ds(t * (RSUB * K) + g * 128, 128)]],
                rows_b.at[pl.ds(g * 128, 128)], sem_b)
        pltpu.async_copy(
            table_h.at[self_v.at[pl.ds(t * RSUB, RSUB)]], srows_b, sem_b)

    def drain(rows_b, srows_b, sem_b):
        pltpu.make_async_copy(
            table_h.at[pl.ds(0, RSUB * K)], rows_b, sem_b).wait()
        pltpu.make_async_copy(
            table_h.at[pl.ds(0, RSUB)], srows_b, sem_b).wait()

    issue(0, rows0, srows0, sem0)

    bufs = (
        (rows0, srows0, out0, sem0, osem0, rows1, srows1, sem1),
        (rows1, srows1, out1, sem1, osem1, rows0, srows0, sem0),
    )

    def outer(u, carry):
        for p, (rb, sb, ob, sm, osm, nrb, nsb, nsm) in enumerate(bufs):
            t = 2 * u + p

            if p == 0:
                issue(t + 1, nrb, nsb, nsm)
            else:
                @pl.when(u < nouter - 1)
                def _():
                    issue(t + 1, nrb, nsb, nsm)

            drain(rb, sb, sm)

            @pl.when(u > 0)
            def _():
                pltpu.make_async_copy(
                    ob, out_h.at[pl.ds(base, RSUB)], osm).wait()

            def body(r, c2):
                row = t * RSUB + r
                wvecs = [w_v[pl.ds(row * s_pad + 16 * g, 16)]
                         for g in range(s_pad // 16)]
                iv = wvecs[(K + 1) // 16][(K + 1) % 16]
                for v in range(nvec):
                    wj = wvecs[K // 16][K % 16]
                    acc = wj * sb[r, pl.ds(v * 16, 16)]
                    for j in range(K):
                        wj = wvecs[j // 16][j % 16]
                        acc = acc + wj * rb[r * K + j, pl.ds(v * 16, 16)]
                    ob[r, pl.ds(v * 16, 16)] = acc * iv
                return c2

            lax.fori_loop(0, RSUB, body, 0)
            pltpu.async_copy(ob, out_h.at[pl.ds(base + t * RSUB, RSUB)], osm)
        return carry

    lax.fori_loop(0, nouter, outer, 0)
    pltpu.make_async_copy(out0, out_h.at[pl.ds(base, RSUB)], osem0).wait()
    pltpu.make_async_copy(out1, out_h.at[pl.ds(base, RSUB)], osem1).wait()


def kernel(table, nodes, to_neighs):
    B, K = to_neighs.shape
    D = table.shape[1]
    S = K + 1

    frac_c0 = 0.0
    tot = -(-B // 16)
    rows_c0 = max(2 * RSUB, int(round(frac_c0 * tot / 16)) * 16)
    if frac_c0 == 0.0:
        rows_c0 = 2 * RSUB
    rows_c1 = max(2 * RSUB, -(-(tot - rows_c0) // 16) * 16)
    b_pad = 16 * (rows_c0 + rows_c1)
    rows_max = max(rows_c0, rows_c1)
    n_alloc = -(-(b_pad + rows_max) // LANE_BLK) * LANE_BLK
    assert b_pad >= B and b_pad % LANE_BLK == 0 and (RSUB * K) % 128 == 0
    neighs = jnp.pad(to_neighs, ((0, n_alloc - B), (0, 0)))
    selfn = jnp.pad(nodes, (0, n_alloc - B))
    samp = jnp.concatenate([neighs, selfn[:, None]], axis=1)

    s_pad = -(-(S + 1) // 16) * 16
    assert n_alloc % LANE_BLK == 0
    s_t = samp.T
    w_t = pl.pallas_call(
        functools.partial(_weights_body, S=S),
        grid=(n_alloc // LANE_BLK,),
        in_specs=[pl.BlockSpec((S, LANE_BLK), lambda i: (0, i))],
        out_specs=[pl.BlockSpec((s_pad, LANE_BLK), lambda i: (0, i))],
        out_shape=[jax.ShapeDtypeStruct((s_pad, n_alloc), jnp.float32)],
    )(s_t)[0]

    idx_flat = neighs.reshape(-1)
    w_flat = w_t.T.reshape(-1)

    mesh = plsc.VectorSubcoreMesh(core_axis_name="c", subcore_axis_name="s")
    sc = pl.kernel(
        functools.partial(_sc_body, K=K, D=D, rows_c0=rows_c0,
                          rows_c1=rows_c1),
        mesh=mesh,
        out_type=jax.ShapeDtypeStruct((b_pad, D), jnp.float32),
        scratch_types=[
            pltpu.VMEM((rows_max * K,), jnp.int32),
            pltpu.VMEM((rows_max,), jnp.int32),
            pltpu.VMEM((rows_max * s_pad,), jnp.float32),
            pltpu.VMEM((RSUB * K, D), jnp.float32),
            pltpu.VMEM((RSUB * K, D), jnp.float32),
            pltpu.VMEM((RSUB, D), jnp.float32),
            pltpu.VMEM((RSUB, D), jnp.float32),
            pltpu.VMEM((RSUB, D), jnp.float32),
            pltpu.VMEM((RSUB, D), jnp.float32),
            pltpu.SemaphoreType.DMA,
            pltpu.SemaphoreType.DMA,
            pltpu.SemaphoreType.DMA,
            pltpu.SemaphoreType.DMA,
        ],
    )
    out = sc(table, idx_flat, selfn, w_flat)
    return out[:B]


# speedup vs baseline: 1.3526x; 1.3526x over previous
import functools

import jax
import jax.numpy as jnp
from jax import lax
from jax.experimental import pallas as pl
from jax.experimental.pallas import tpu as pltpu
from jax.experimental.pallas import tpu_sc as plsc

NW = 32
RSUB = 8
LANE_BLK = 256


def _weights_body(s_ref, w_ref, *, S):
    s = s_ref[...]
    c = jnp.zeros(s.shape, jnp.float32)
    for j in range(S):
        c = c + (s == s[j : j + 1, :]).astype(jnp.float32)
    w = 1.0 / c
    w_ref[: S, :] = w
    w_ref[S : S + 1, :] = 1.0 / jnp.sum(w, axis=0, keepdims=True)


def _sc_body(table_h, idx_h, self_h, w_h, out_h,
             idx_v, self_v, w_v,
             rows0, rows1, srows0, srows1, out0, out1,
             sem0, sem1, osem0, osem1,
             *, K, D, rows_c0, rows_c1):
    c = lax.axis_index("c")
    s = lax.axis_index("s")
    rows_mine = lax.select(c == 0, rows_c0, rows_c1)
    base = pl.multiple_of(c * (16 * rows_c0) + s * rows_mine, 2 * RSUB)
    rows_max = max(rows_c0, rows_c1)
    s_pad = -(-(K + 2) // 16) * 16
    pltpu.sync_copy(idx_h.at[pl.ds(base * K, rows_max * K)],
                    idx_v.at[pl.ds(0, rows_max * K)])
    pltpu.sync_copy(self_h.at[pl.ds(base, rows_max)],
                    self_v.at[pl.ds(0, rows_max)])
    pltpu.sync_copy(w_h.at[pl.ds(base * s_pad, rows_max * s_pad)],
                    w_v.at[pl.ds(0, rows_max * s_pad)])

    nvec = D // 16
    nstream = (RSUB * K) // 128
    nsteps = rows_mine // RSUB
    nouter = nsteps // 2

    def issue(t, rows_b, srows_b, sem_b):
        for g in range(nstream * 2):
            pltpu.async_copy(
                table_h.at[idx_v.at[pl.ds(t * (RSUB * K) + g * 64, 64)]],
                rows_b.at[pl.ds(g * 64, 64)], sem_b)
        pltpu.async_copy(
            table_h.at[self_v.at[pl.ds(t * RSUB, RSUB)]], srows_b, sem_b)

    def drain(rows_b, srows_b, sem_b):
        pltpu.make_async_copy(
            table_h.at[pl.ds(0, RSUB * K)], rows_b, sem_b).wait()
        pltpu.make_async_copy(
            table_h.at[pl.ds(0, RSUB)], srows_b, sem_b).wait()

    issue(0, rows0, srows0, sem0)

    bufs = (
        (rows0, srows0, out0, sem0, osem0, rows1, srows1, sem1),
        (rows1, srows1, out1, sem1, osem1, rows0, srows0, sem0),
    )

    def outer(u, carry):
        for p, (rb, sb, ob, sm, osm, nrb, nsb, nsm) in enumerate(bufs):
            t = 2 * u + p

            if p == 0:
                issue(t + 1, nrb, nsb, nsm)
            else:
                @pl.when(u < nouter - 1)
                def _():
                    issue(t + 1, nrb, nsb, nsm)

            drain(rb, sb, sm)

            @pl.when(u > 0)
            def _():
                pltpu.make_async_copy(
                    ob, out_h.at[pl.ds(base, RSUB)], osm).wait()

            def body(r, c2):
                row = t * RSUB + r
                wvecs = [w_v[pl.ds(row * s_pad + 16 * g, 16)]
                         for g in range(s_pad // 16)]
                iv = wvecs[(K + 1) // 16][(K + 1) % 16]
                for v in range(nvec):
                    wj = wvecs[K // 16][K % 16]
                    acc = wj * sb[r, pl.ds(v * 16, 16)]
                    for j in range(K):
                        wj = wvecs[j // 16][j % 16]
                        acc = acc + wj * rb[r * K + j, pl.ds(v * 16, 16)]
                    ob[r, pl.ds(v * 16, 16)] = acc * iv
                return c2

            lax.fori_loop(0, RSUB, body, 0)
            pltpu.async_copy(ob, out_h.at[pl.ds(base + t * RSUB, RSUB)], osm)
        return carry

    lax.fori_loop(0, nouter, outer, 0)
    pltpu.make_async_copy(out0, out_h.at[pl.ds(base, RSUB)], osem0).wait()
    pltpu.make_async_copy(out1, out_h.at[pl.ds(base, RSUB)], osem1).wait()


def kernel(table, nodes, to_neighs):
    B, K = to_neighs.shape
    D = table.shape[1]
    S = K + 1

    frac_c0 = 1.0
    tot = -(-B // 16)
    rows_c0 = max(2 * RSUB, int(round(frac_c0 * tot / 16)) * 16)
    rows_c1 = max(2 * RSUB, -(-(tot - rows_c0) // 16) * 16)
    b_pad = 16 * (rows_c0 + rows_c1)
    rows_max = max(rows_c0, rows_c1)
    n_alloc = -(-(b_pad + rows_max) // LANE_BLK) * LANE_BLK
    assert b_pad >= B and b_pad % LANE_BLK == 0 and (RSUB * K) % 128 == 0
    neighs = jnp.pad(to_neighs, ((0, n_alloc - B), (0, 0)))
    selfn = jnp.pad(nodes, (0, n_alloc - B))
    samp = jnp.concatenate([neighs, selfn[:, None]], axis=1)

    s_pad = -(-(S + 1) // 16) * 16
    assert n_alloc % LANE_BLK == 0
    s_t = samp.T
    w_t = pl.pallas_call(
        functools.partial(_weights_body, S=S),
        grid=(n_alloc // LANE_BLK,),
        in_specs=[pl.BlockSpec((S, LANE_BLK), lambda i: (0, i))],
        out_specs=[pl.BlockSpec((s_pad, LANE_BLK), lambda i: (0, i))],
        out_shape=[jax.ShapeDtypeStruct((s_pad, n_alloc), jnp.float32)],
    )(s_t)[0]

    idx_flat = neighs.reshape(-1)
    w_flat = w_t.T.reshape(-1)

    mesh = plsc.VectorSubcoreMesh(core_axis_name="c", subcore_axis_name="s")
    sc = pl.kernel(
        functools.partial(_sc_body, K=K, D=D, rows_c0=rows_c0,
                          rows_c1=rows_c1),
        mesh=mesh,
        out_type=jax.ShapeDtypeStruct((b_pad, D), jnp.float32),
        scratch_types=[
            pltpu.VMEM((rows_max * K,), jnp.int32),
            pltpu.VMEM((rows_max,), jnp.int32),
            pltpu.VMEM((rows_max * s_pad,), jnp.float32),
            pltpu.VMEM((RSUB * K, D), jnp.float32),
            pltpu.VMEM((RSUB * K, D), jnp.float32),
            pltpu.VMEM((RSUB, D), jnp.float32),
            pltpu.VMEM((RSUB, D), jnp.float32),
            pltpu.VMEM((RSUB, D), jnp.float32),
            pltpu.VMEM((RSUB, D), jnp.float32),
            pltpu.SemaphoreType.DMA,
            pltpu.SemaphoreType.DMA,
            pltpu.SemaphoreType.DMA,
            pltpu.SemaphoreType.DMA,
        ],
    )
    out = sc(table, idx_flat, selfn, w_flat)
    return out[:B]


# speedup vs baseline: 1.3548x; 1.0016x over previous
import functools

import jax
import jax.numpy as jnp
from jax import lax
from jax.experimental import pallas as pl
from jax.experimental.pallas import tpu as pltpu
from jax.experimental.pallas import tpu_sc as plsc

NW = 32
RSUB = 8
LANE_BLK = 256


def _weights_body(s_ref, w_ref, *, S):
    s = s_ref[...]
    c = jnp.zeros(s.shape, jnp.float32)
    for j in range(S):
        c = c + (s == s[j : j + 1, :]).astype(jnp.float32)
    w = 1.0 / c
    w_ref[: S, :] = w
    w_ref[S : S + 1, :] = 1.0 / jnp.sum(w, axis=0, keepdims=True)


def _sc_body(table_h, idx_h, self_h, w_h, out_h,
             idx_v, self_v, w_v,
             rows0, rows1, srows0, srows1, out0, out1,
             sem0, sem1, osem0, osem1,
             *, K, D, rows_c0, rows_c1):
    c = lax.axis_index("c")
    s = lax.axis_index("s")
    rows_mine = lax.select(c == 0, rows_c0, rows_c1)
    base = pl.multiple_of(c * (16 * rows_c0) + s * rows_mine, 2 * RSUB)
    rows_max = max(rows_c0, rows_c1)
    s_pad = -(-(K + 2) // 16) * 16
    pltpu.sync_copy(idx_h.at[pl.ds(base * K, rows_max * K)],
                    idx_v.at[pl.ds(0, rows_max * K)])
    pltpu.sync_copy(self_h.at[pl.ds(base, rows_max)],
                    self_v.at[pl.ds(0, rows_max)])
    pltpu.sync_copy(w_h.at[pl.ds(base * s_pad, rows_max * s_pad)],
                    w_v.at[pl.ds(0, rows_max * s_pad)])

    nvec = D // 16
    nstream = (RSUB * K) // 128
    nsteps = rows_mine // RSUB
    nouter = nsteps // 2

    def issue(t, rows_b, srows_b, sem_b):
        for g in range(nstream):
            pltpu.async_copy(
                table_h.at[idx_v.at[pl.ds(t * (RSUB * K) + g * 128, 128)]],
                rows_b.at[pl.ds(g * 128, 128)], sem_b)
        pltpu.async_copy(
            table_h.at[self_v.at[pl.ds(t * RSUB, RSUB)]], srows_b, sem_b)

    def drain(rows_b, srows_b, sem_b):
        pltpu.make_async_copy(
            table_h.at[pl.ds(0, RSUB * K)], rows_b, sem_b).wait()
        pltpu.make_async_copy(
            table_h.at[pl.ds(0, RSUB)], srows_b, sem_b).wait()

    issue(0, rows0, srows0, sem0)

    bufs = (
        (rows0, srows0, out0, sem0, osem0, rows1, srows1, sem1),
        (rows1, srows1, out1, sem1, osem1, rows0, srows0, sem0),
    )

    def outer(u, carry):
        for p, (rb, sb, ob, sm, osm, nrb, nsb, nsm) in enumerate(bufs):
            t = 2 * u + p

            if p == 0:
                issue(t + 1, nrb, nsb, nsm)
            else:
                @pl.when(u < nouter - 1)
                def _():
                    issue(t + 1, nrb, nsb, nsm)

            drain(rb, sb, sm)

            @pl.when(u > 0)
            def _():
                pltpu.make_async_copy(
                    ob, out_h.at[pl.ds(base, RSUB)], osm).wait()

            def body(r, c2):
                row = t * RSUB + r
                wvecs = [w_v[pl.ds(row * s_pad + 16 * g, 16)]
                         for g in range(s_pad // 16)]
                iv = wvecs[(K + 1) // 16][(K + 1) % 16]
                for v in range(nvec):
                    wj = wvecs[K // 16][K % 16]
                    acc = wj * sb[r, pl.ds(v * 16, 16)]
                    for j in range(K):
                        wj = wvecs[j // 16][j % 16]
                        acc = acc + wj * rb[r * K + j, pl.ds(v * 16, 16)]
                    ob[r, pl.ds(v * 16, 16)] = acc * iv
                return c2

            lax.fori_loop(0, RSUB, body, 0)
            pltpu.async_copy(ob, out_h.at[pl.ds(base + t * RSUB, RSUB)], osm)
        return carry

    lax.fori_loop(0, nouter, outer, 0)
    pltpu.make_async_copy(out0, out_h.at[pl.ds(base, RSUB)], osem0).wait()
    pltpu.make_async_copy(out1, out_h.at[pl.ds(base, RSUB)], osem1).wait()


def kernel(table, nodes, to_neighs):
    B, K = to_neighs.shape
    D = table.shape[1]
    S = K + 1

    frac_c0 = 1.0
    tot = -(-B // 16)
    rows_c0 = max(2 * RSUB, int(round(frac_c0 * tot / 16)) * 16)
    rows_c1 = max(2 * RSUB, -(-(tot - rows_c0) // 16) * 16)
    b_pad = 16 * (rows_c0 + rows_c1)
    rows_max = max(rows_c0, rows_c1)
    n_alloc = -(-(b_pad + rows_max) // LANE_BLK) * LANE_BLK
    assert b_pad >= B and b_pad % LANE_BLK == 0 and (RSUB * K) % 128 == 0
    neighs = jnp.pad(to_neighs, ((0, n_alloc - B), (0, 0)))
    selfn = jnp.pad(nodes, (0, n_alloc - B))
    samp = jnp.concatenate([neighs, selfn[:, None]], axis=1)

    s_pad = -(-(S + 1) // 16) * 16
    assert n_alloc % LANE_BLK == 0
    s_t = samp.T
    w_t = pl.pallas_call(
        functools.partial(_weights_body, S=S),
        grid=(n_alloc // LANE_BLK,),
        in_specs=[pl.BlockSpec((S, LANE_BLK), lambda i: (0, i))],
        out_specs=[pl.BlockSpec((s_pad, LANE_BLK), lambda i: (0, i))],
        out_shape=[jax.ShapeDtypeStruct((s_pad, n_alloc), jnp.float32)],
    )(s_t)[0]

    idx_flat = neighs.reshape(-1)
    w_flat = w_t.T.reshape(-1)

    mesh = plsc.VectorSubcoreMesh(core_axis_name="c", subcore_axis_name="s")
    sc = pl.kernel(
        functools.partial(_sc_body, K=K, D=D, rows_c0=rows_c0,
                          rows_c1=rows_c1),
        mesh=mesh,
        out_type=jax.ShapeDtypeStruct((b_pad, D), jnp.float32),
        scratch_types=[
            pltpu.VMEM((rows_max * K,), jnp.int32),
            pltpu.VMEM((rows_max,), jnp.int32),
            pltpu.VMEM((rows_max * s_pad,), jnp.float32),
            pltpu.VMEM((RSUB * K, D), jnp.float32),
            pltpu.VMEM((RSUB * K, D), jnp.float32),
            pltpu.VMEM((RSUB, D), jnp.float32),
            pltpu.VMEM((RSUB, D), jnp.float32),
            pltpu.VMEM((RSUB, D), jnp.float32),
            pltpu.VMEM((RSUB, D), jnp.float32),
            pltpu.SemaphoreType.DMA,
            pltpu.SemaphoreType.DMA,
            pltpu.SemaphoreType.DMA,
            pltpu.SemaphoreType.DMA,
        ],
    )
    out = sc(table, idx_flat, selfn, w_flat)
    return out[:B]
